# Initial kernel scaffold; baseline (speedup 1.0000x reference)
import jax
import jax.numpy as jnp
from jax.experimental import pallas as pl


def kernel(feat_Matrix, X_Node, X_Neis, dg_list, W_xi, b_xi, W_rou, b_rou, W_out, b_out):
    raise NotImplementedError("write your pallas kernel here")


# speedup vs baseline: 1.8653x; 1.8653x over previous
import functools

import jax
import jax.numpy as jnp
from jax import lax
from jax.experimental import pallas as pl
from jax.experimental.pallas import tpu as pltpu
from jax.experimental.pallas import tpu_sc as plsc

V = 10000
E = 160000
LN = 128
S = 32
C = 40
MU = 0.9

NC = 2
NS = 16
NW = NC * NS

CH = 128
E_PAD = 163840
PER_W = E_PAD // NW
N_CH = PER_W // CH
V_PAD = 10112
ROWS_PER_TILE = V_PAD // NS

_MESH = dict(core_axis_name="c", subcore_axis_name="s")


def _sc_gather_embeds(feat_pad, xn, xe):
    @functools.partial(
        pl.kernel,
        mesh=plsc.VectorSubcoreMesh(**_MESH),
        compiler_params=pltpu.CompilerParams(use_tc_tiling_on_sc=False),
        out_type=(jax.ShapeDtypeStruct((E_PAD, LN), jnp.float32),
                  jax.ShapeDtypeStruct((E_PAD, LN), jnp.float32)),
        scratch_types=[pltpu.VMEM((CH,), jnp.int32),
                       pltpu.VMEM((CH, LN), jnp.float32),
                       pltpu.SemaphoreType.DMA],
    )
    def k(feat_hbm, xn_hbm, xe_hbm, node_out, neis_out, idx_v, rows_v, sem):
        wid = lax.axis_index("s") * NC + lax.axis_index("c")
        base = wid * PER_W

        def chunk(i, idx_hbm, out_hbm):
            off = base + i * CH
            pltpu.sync_copy(idx_hbm.at[pl.ds(off, CH)], idx_v)
            pltpu.async_copy(feat_hbm.at[idx_v], rows_v, sem).wait()
            pltpu.sync_copy(rows_v, out_hbm.at[pl.ds(off, CH)])

        def loop_n(i, carry):
            chunk(i, xn_hbm, node_out)
            return carry

        def loop_e(i, carry):
            chunk(i, xe_hbm, neis_out)
            return carry

        lax.fori_loop(0, N_CH, loop_n, 0)
        lax.fori_loop(0, N_CH, loop_e, 0)

    return k(feat_pad, xn, xe)


def _sc_scatter_add(vals, idx, zeros_tile):
    @functools.partial(
        pl.kernel,
        mesh=plsc.VectorSubcoreMesh(**_MESH),
        compiler_params=pltpu.CompilerParams(use_tc_tiling_on_sc=False),
        out_type=jax.ShapeDtypeStruct((NC, V_PAD, S), jnp.float32),
        scratch_types=[pltpu.VMEM((CH,), jnp.int32),
                       pltpu.VMEM((CH, S), jnp.float32),
                       pltpu.VMEM_SHARED((V_PAD, S), jnp.float32),
                       pltpu.SemaphoreType.DMA],
    )
    def k(vals_hbm, idx_hbm, zeros_hbm, out_hbm, idx_v, vals_v, h_sh, sem):
        cid = lax.axis_index("c")
        sid = lax.axis_index("s")
        wid = sid * NC + cid
        base = wid * PER_W

        pltpu.sync_copy(zeros_hbm, h_sh.at[pl.ds(sid * ROWS_PER_TILE, ROWS_PER_TILE)])
        plsc.subcore_barrier()

        def loop(i, carry):
            off = base + i * CH
            pltpu.sync_copy(idx_hbm.at[pl.ds(off, CH)], idx_v)
            pltpu.sync_copy(vals_hbm.at[pl.ds(off, CH)], vals_v)
            pltpu.sync_copy(vals_v, h_sh.at[idx_v], add=True)
            return carry

        lax.fori_loop(0, N_CH, loop, 0)
        plsc.subcore_barrier()

        pltpu.sync_copy(h_sh.at[pl.ds(sid * ROWS_PER_TILE, ROWS_PER_TILE)],
                        out_hbm.at[cid, pl.ds(sid * ROWS_PER_TILE, ROWS_PER_TILE)])

    return k(vals, idx, zeros_tile)


def _sc_gather_h(q0, q1, xn):
    @functools.partial(
        pl.kernel,
        mesh=plsc.VectorSubcoreMesh(**_MESH),
        compiler_params=pltpu.CompilerParams(use_tc_tiling_on_sc=False),
        out_type=(jax.ShapeDtypeStruct((E_PAD, S), jnp.float32),
                  jax.ShapeDtypeStruct((E_PAD, S), jnp.float32)),
        scratch_types=[pltpu.VMEM((CH,), jnp.int32),
                       pltpu.VMEM((CH, S), jnp.float32),
                       pltpu.SemaphoreType.DMA],
    )
    def k(q0_hbm, q1_hbm, xn_hbm, hg0_out, hg1_out, idx_v, rows_v, sem):
        wid = lax.axis_index("s") * NC + lax.axis_index("c")
        base = wid * PER_W

        def loop(i, carry):
            off = base + i * CH
            pltpu.sync_copy(xn_hbm.at[pl.ds(off, CH)], idx_v)
            pltpu.async_copy(q0_hbm.at[idx_v], rows_v, sem).wait()
            pltpu.sync_copy(rows_v, hg0_out.at[pl.ds(off, CH)])
            pltpu.async_copy(q1_hbm.at[idx_v], rows_v, sem).wait()
            pltpu.sync_copy(rows_v, hg1_out.at[pl.ds(off, CH)])
            return carry

        lax.fori_loop(0, N_CH, loop, 0)

    return k(q0, q1, xn)


def _b_body(node_ref, w_ref, brou_ref, out_ref):
    z = jnp.dot(node_ref[...], w_ref[...], preferred_element_type=jnp.float32)
    out_ref[...] = jnp.tanh(z + brou_ref[...])


def _tc_b(node_e, w_rou_t, b_rou):
    BE = 2048
    return pl.pallas_call(
        _b_body,
        grid=(E_PAD // BE,),
        in_specs=[pl.BlockSpec((BE, LN), lambda i: (i, 0)),
                  pl.BlockSpec((LN, S), lambda i: (0, 0)),
                  pl.BlockSpec((1, S), lambda i: (0, 0))],
        out_specs=pl.BlockSpec((BE, S), lambda i: (i, 0)),
        out_shape=jax.ShapeDtypeStruct((E_PAD, S), jnp.float32),
    )(node_e, w_rou_t, b_rou)


def _he_body(node_ref, neis_ref, hg0_ref, hg1_ref, b_ref, dg_ref,
             w1_ref, w2_ref, bxi_ref, q2_ref, r2_ref, out_ref):
    z = jnp.dot(node_ref[...], w1_ref[...], preferred_element_type=jnp.float32)
    z = z + jnp.dot(neis_ref[...], w2_ref[...], preferred_element_type=jnp.float32)
    a = jnp.tanh(z + bxi_ref[...])
    hg = hg0_ref[...] + hg1_ref[...]
    hrep = jnp.dot(hg, q2_ref[...], preferred_element_type=jnp.float32)
    he = jnp.dot(a * hrep, r2_ref[...], preferred_element_type=jnp.float32)
    out_ref[...] = he * ((MU / S) / dg_ref[...]) + b_ref[...]


def _tc_he(node_e, neis_e, hg0, hg1, b_mat, dg, w1, w2, b_xi, q2c, r2c):
    BE = 512
    return pl.pallas_call(
        _he_body,
        grid=(E_PAD // BE,),
        in_specs=[pl.BlockSpec((BE, LN), lambda i: (i, 0)),
                  pl.BlockSpec((BE, LN), lambda i: (i, 0)),
                  pl.BlockSpec((BE, S), lambda i: (i, 0)),
                  pl.BlockSpec((BE, S), lambda i: (i, 0)),
                  pl.BlockSpec((BE, S), lambda i: (i, 0)),
                  pl.BlockSpec((BE, 1), lambda i: (i, 0)),
                  pl.BlockSpec((LN, S * S), lambda i: (0, 0)),
                  pl.BlockSpec((LN, S * S), lambda i: (0, 0)),
                  pl.BlockSpec((1, S * S), lambda i: (0, 0)),
                  pl.BlockSpec((S, S * S), lambda i: (0, 0)),
                  pl.BlockSpec((S * S, S), lambda i: (0, 0))],
        out_specs=pl.BlockSpec((BE, S), lambda i: (i, 0)),
        out_shape=jax.ShapeDtypeStruct((E_PAD, S), jnp.float32),
    )(node_e, neis_e, hg0, hg1, b_mat, dg, w1, w2, b_xi, q2c, r2c)


def _out_body(q0_ref, q1_ref, w_ref, bout_ref, out_ref):
    h = q0_ref[...] + q1_ref[...]
    logits = jnp.dot(h, w_ref[...], preferred_element_type=jnp.float32)
    logits = logits + bout_ref[...]
    m = jnp.max(logits, axis=-1, keepdims=True)
    lse = jnp.log(jnp.sum(jnp.exp(logits - m), axis=-1, keepdims=True)) + m
    out_ref[...] = logits - lse


def _tc_out(q0, q1, w_out_t, b_out):
    return pl.pallas_call(
        _out_body,
        grid=(1,),
        in_specs=[pl.BlockSpec((V, S), lambda i: (0, 0)),
                  pl.BlockSpec((V, S), lambda i: (0, 0)),
                  pl.BlockSpec((S, C), lambda i: (0, 0)),
                  pl.BlockSpec((1, C), lambda i: (0, 0))],
        out_specs=pl.BlockSpec((V, C), lambda i: (0, 0)),
        out_shape=jax.ShapeDtypeStruct((V, C), jnp.float32),
    )(q0, q1, w_out_t, b_out)


def kernel(feat_Matrix, X_Node, X_Neis, dg_list, W_xi, b_xi, W_rou, b_rou,
           W_out, b_out):
    f32 = jnp.float32
    feat_pad = jnp.pad(feat_Matrix.astype(f32), ((0, V_PAD - V), (0, 0)))
    xn = jnp.pad(X_Node.astype(jnp.int32), (0, E_PAD - E), constant_values=V)
    xe = jnp.pad(X_Neis.astype(jnp.int32), (0, E_PAD - E), constant_values=V)
    dg = jnp.pad(dg_list.astype(f32), (0, E_PAD - E),
                 constant_values=1.0).reshape(E_PAD, 1)

    w1 = W_xi[:, :LN].T.astype(f32)
    w2 = W_xi[:, LN:].T.astype(f32)
    bxi = b_xi.reshape(1, S * S).astype(f32)
    w_rou_t = W_rou.T.astype(f32)
    brou = b_rou.reshape(1, S).astype(f32)
    w_out_t = W_out.T.astype(f32)
    bout = b_out.reshape(1, C).astype(f32)
    eye = jnp.eye(S, dtype=f32)
    q2c = jnp.tile(eye, (1, S))
    r2c = jnp.repeat(eye, S, axis=0)
    zeros_tile = jnp.zeros((ROWS_PER_TILE, S), dtype=f32)

    node_e, neis_e = _sc_gather_embeds(feat_pad, xn, xe)
    b_mat = _tc_b(node_e, w_rou_t, brou)
    q = _sc_scatter_add(b_mat, xn, zeros_tile)
    hg0, hg1 = _sc_gather_h(q[0], q[1], xn)
    he = _tc_he(node_e, neis_e, hg0, hg1, b_mat, dg, w1, w2, bxi, q2c, r2c)
    q2 = _sc_scatter_add(he, xn, zeros_tile)
    return _tc_out(q2[0, :V], q2[1, :V], w_out_t, bout)


# speedup vs baseline: 2.5297x; 1.3562x over previous
import functools

import jax
import jax.numpy as jnp
from jax import lax
from jax.experimental import pallas as pl
from jax.experimental.pallas import tpu as pltpu
from jax.experimental.pallas import tpu_sc as plsc

V = 10000
E = 160000
LN = 128
S = 32
C = 40
MU = 0.9

NC = 2
NS = 16
NW = NC * NS

CH = 128
E_PAD = 163840
PER_W = E_PAD // NW
N_CH = PER_W // CH
PER_T = E_PAD // NS
N_CHT = PER_T // CH
V_PAD = 10112
ROWS_PER_TILE = V_PAD // NS
NB = 4

_MESH = dict(core_axis_name="c", subcore_axis_name="s")
_SC_PARAMS = dict(
    mesh=plsc.VectorSubcoreMesh(**_MESH),
    compiler_params=pltpu.CompilerParams(use_tc_tiling_on_sc=False),
)


def _sc_gather_embeds(feat_pad, xn3, xe3):
    @functools.partial(
        pl.kernel,
        **_SC_PARAMS,
        out_type=(jax.ShapeDtypeStruct((E_PAD, LN), jnp.float32),
                  jax.ShapeDtypeStruct((E_PAD, LN), jnp.float32)),
        scratch_types=[pltpu.VMEM((N_CH, CH), jnp.int32),
                       pltpu.VMEM((N_CH, CH), jnp.int32),
                       pltpu.VMEM((NB, CH, LN), jnp.float32),
                       pltpu.SemaphoreType.DMA((NB,)),
                       pltpu.SemaphoreType.DMA((NB,))],
    )
    def k(feat_hbm, xn_hbm, xe_hbm, node_out, neis_out,
          idxn_v, idxe_v, ring, gsem, wsem):
        wid = lax.axis_index("s") * NC + lax.axis_index("c")
        base = wid * PER_W
        pltpu.sync_copy(xn_hbm.at[wid], idxn_v)
        pltpu.sync_copy(xe_hbm.at[wid], idxe_v)

        def run(idx_v, out_hbm):
            for j in range(NB):
                pltpu.async_copy(feat_hbm.at[idx_v.at[j]], ring.at[j],
                                 gsem.at[j])

            def body(r, carry):
                for j in range(NB):
                    i = r * NB + j
                    pltpu.make_async_copy(feat_hbm.at[idx_v.at[j]],
                                          ring.at[j], gsem.at[j]).wait()
                    pltpu.async_copy(ring.at[j],
                                     out_hbm.at[pl.ds(base + i * CH, CH)],
                                     wsem.at[j])
                for j in range(NB):
                    i2 = (r + 1) * NB + j

                    @pl.when(i2 < N_CH)
                    def _():
                        pltpu.make_async_copy(
                            ring.at[j], out_hbm.at[pl.ds(0, CH)],
                            wsem.at[j]).wait()
                        pltpu.async_copy(feat_hbm.at[idx_v.at[i2]],
                                         ring.at[j], gsem.at[j])
                return carry

            lax.fori_loop(0, N_CH // NB, body, 0)
            for j in range(NB):
                pltpu.make_async_copy(ring.at[j], out_hbm.at[pl.ds(0, CH)],
                                      wsem.at[j]).wait()

        run(idxn_v, node_out)
        run(idxe_v, neis_out)

    return k(feat_pad, xn3, xe3)


def _sc_segsum_gather(vals, idx3, zeros_tile):
    @functools.partial(
        pl.kernel,
        **_SC_PARAMS,
        out_type=jax.ShapeDtypeStruct((E_PAD, S), jnp.float32),
        scratch_types=[pltpu.VMEM((N_CHT, CH), jnp.int32),
                       pltpu.VMEM((NB, CH, S), jnp.float32),
                       pltpu.VMEM_SHARED((V_PAD, S), jnp.float32),
                       pltpu.SemaphoreType.DMA((NB,)),
                       pltpu.SemaphoreType.DMA((NB,))],
    )
    def k(vals_hbm, idx_hbm, zeros_hbm, hg_out, idx_v, ring, h_sh, lsem, ssem):
        cid = lax.axis_index("c")
        sid = lax.axis_index("s")
        tbase = sid * PER_T

        pltpu.sync_copy(idx_hbm.at[sid], idx_v)
        pltpu.sync_copy(zeros_hbm,
                        h_sh.at[pl.ds(sid * ROWS_PER_TILE, ROWS_PER_TILE)])
        plsc.subcore_barrier()

        for j in range(NB):
            pltpu.async_copy(vals_hbm.at[pl.ds(tbase + j * CH, CH)],
                             ring.at[j], lsem.at[j])

        def body(r, carry):
            for j in range(NB):
                i = r * NB + j
                pltpu.make_async_copy(
                    vals_hbm.at[pl.ds(tbase + i * CH, CH)], ring.at[j],
                    lsem.at[j]).wait()
                pltpu.async_copy(ring.at[j], h_sh.at[idx_v.at[i]],
                                 ssem.at[j], add=True)
            for j in range(NB):
                i2 = (r + 1) * NB + j

                @pl.when(i2 < N_CHT)
                def _():
                    pltpu.make_async_copy(ring.at[j],
                                          h_sh.at[pl.ds(0, CH)],
                                          ssem.at[j]).wait()
                    pltpu.async_copy(
                        vals_hbm.at[pl.ds(tbase + i2 * CH, CH)], ring.at[j],
                        lsem.at[j])
            return carry

        lax.fori_loop(0, N_CHT // NB, body, 0)
        for j in range(NB):
            pltpu.make_async_copy(ring.at[j], h_sh.at[pl.ds(0, CH)],
                                  ssem.at[j]).wait()
        plsc.subcore_barrier()

        wbase = tbase + cid * PER_W
        row0 = cid * N_CH

        for j in range(NB):
            pltpu.async_copy(h_sh.at[idx_v.at[row0 + j]], ring.at[j],
                             lsem.at[j])

        def gbody(r, carry):
            for j in range(NB):
                i = r * NB + j
                pltpu.make_async_copy(h_sh.at[idx_v.at[row0 + i]],
                                      ring.at[j], lsem.at[j]).wait()
                pltpu.async_copy(ring.at[j],
                                 hg_out.at[pl.ds(wbase + i * CH, CH)],
                                 ssem.at[j])
            for j in range(NB):
                i2 = (r + 1) * NB + j

                @pl.when(i2 < N_CH)
                def _():
                    pltpu.make_async_copy(ring.at[j],
                                          hg_out.at[pl.ds(0, CH)],
                                          ssem.at[j]).wait()
                    pltpu.async_copy(h_sh.at[idx_v.at[row0 + i2]],
                                     ring.at[j], lsem.at[j])
            return carry

        lax.fori_loop(0, N_CH // NB, gbody, 0)
        for j in range(NB):
            pltpu.make_async_copy(ring.at[j], hg_out.at[pl.ds(0, CH)],
                                  ssem.at[j]).wait()

    return k(vals, idx3, zeros_tile)


def _sc_scatter_add(vals, idx3, zeros_tile):
    @functools.partial(
        pl.kernel,
        **_SC_PARAMS,
        out_type=jax.ShapeDtypeStruct((NC, V_PAD, S), jnp.float32),
        scratch_types=[pltpu.VMEM((N_CHT, CH), jnp.int32),
                       pltpu.VMEM((NB, CH, S), jnp.float32),
                       pltpu.VMEM_SHARED((V_PAD, S), jnp.float32),
                       pltpu.SemaphoreType.DMA((NB,)),
                       pltpu.SemaphoreType.DMA((NB,))],
    )
    def k(vals_hbm, idx_hbm, zeros_hbm, out_hbm, idx_v, ring, h_sh,
          lsem, ssem):
        cid = lax.axis_index("c")
        sid = lax.axis_index("s")
        wbase = sid * PER_T + cid * PER_W
        row0 = cid * N_CH

        pltpu.sync_copy(idx_hbm.at[sid], idx_v)
        pltpu.sync_copy(zeros_hbm,
                        h_sh.at[pl.ds(sid * ROWS_PER_TILE, ROWS_PER_TILE)])
        plsc.subcore_barrier()

        for j in range(NB):
            pltpu.async_copy(vals_hbm.at[pl.ds(wbase + j * CH, CH)],
                             ring.at[j], lsem.at[j])

        def body(r, carry):
            for j in range(NB):
                i = r * NB + j
                pltpu.make_async_copy(
                    vals_hbm.at[pl.ds(wbase + i * CH, CH)], ring.at[j],
                    lsem.at[j]).wait()
                pltpu.async_copy(ring.at[j], h_sh.at[idx_v.at[row0 + i]],
                                 ssem.at[j], add=True)
            for j in range(NB):
                i2 = (r + 1) * NB + j

                @pl.when(i2 < N_CH)
                def _():
                    pltpu.make_async_copy(ring.at[j],
                                          h_sh.at[pl.ds(0, CH)],
                                          ssem.at[j]).wait()
                    pltpu.async_copy(
                        vals_hbm.at[pl.ds(wbase + i2 * CH, CH)], ring.at[j],
                        lsem.at[j])
            return carry

        lax.fori_loop(0, N_CH // NB, body, 0)
        for j in range(NB):
            pltpu.make_async_copy(ring.at[j], h_sh.at[pl.ds(0, CH)],
                                  ssem.at[j]).wait()
        plsc.subcore_barrier()

        pltpu.sync_copy(h_sh.at[pl.ds(sid * ROWS_PER_TILE, ROWS_PER_TILE)],
                        out_hbm.at[cid, pl.ds(sid * ROWS_PER_TILE,
                                              ROWS_PER_TILE)])

    return k(vals, idx3, zeros_tile)


def _b_body(node_ref, w_ref, brou_ref, out_ref):
    z = jnp.dot(node_ref[...], w_ref[...], preferred_element_type=jnp.float32)
    out_ref[...] = jnp.tanh(z + brou_ref[...])


def _tc_b(node_e, w_rou_t, b_rou):
    BE = 2048
    return pl.pallas_call(
        _b_body,
        grid=(E_PAD // BE,),
        in_specs=[pl.BlockSpec((BE, LN), lambda i: (i, 0)),
                  pl.BlockSpec((LN, S), lambda i: (0, 0)),
                  pl.BlockSpec((1, S), lambda i: (0, 0))],
        out_specs=pl.BlockSpec((BE, S), lambda i: (i, 0)),
        out_shape=jax.ShapeDtypeStruct((E_PAD, S), jnp.float32),
    )(node_e, w_rou_t, b_rou)


def _he_body(node_ref, neis_ref, hg_ref, b_ref, dg_ref,
             w1_ref, w2_ref, bxi_ref, q2_ref, r2_ref, out_ref):
    z = jnp.dot(node_ref[...], w1_ref[...], preferred_element_type=jnp.float32)
    z = z + jnp.dot(neis_ref[...], w2_ref[...], preferred_element_type=jnp.float32)
    a = jnp.tanh(z + bxi_ref[...])
    hrep = jnp.dot(hg_ref[...], q2_ref[...], preferred_element_type=jnp.float32)
    he = jnp.dot(a * hrep, r2_ref[...], preferred_element_type=jnp.float32)
    out_ref[...] = he * ((MU / S) / dg_ref[...]) + b_ref[...]


def _tc_he(node_e, neis_e, hg, b_mat, dg, w1, w2, b_xi, q2c, r2c):
    BE = 512
    return pl.pallas_call(
        _he_body,
        grid=(E_PAD // BE,),
        in_specs=[pl.BlockSpec((BE, LN), lambda i: (i, 0)),
                  pl.BlockSpec((BE, LN), lambda i: (i, 0)),
                  pl.BlockSpec((BE, S), lambda i: (i, 0)),
                  pl.BlockSpec((BE, S), lambda i: (i, 0)),
                  pl.BlockSpec((BE, 1), lambda i: (i, 0)),
                  pl.BlockSpec((LN, S * S), lambda i: (0, 0)),
                  pl.BlockSpec((LN, S * S), lambda i: (0, 0)),
                  pl.BlockSpec((1, S * S), lambda i: (0, 0)),
                  pl.BlockSpec((S, S * S), lambda i: (0, 0)),
                  pl.BlockSpec((S * S, S), lambda i: (0, 0))],
        out_specs=pl.BlockSpec((BE, S), lambda i: (i, 0)),
        out_shape=jax.ShapeDtypeStruct((E_PAD, S), jnp.float32),
    )(node_e, neis_e, hg, b_mat, dg, w1, w2, b_xi, q2c, r2c)


def _out_body(q0_ref, q1_ref, w_ref, bout_ref, out_ref):
    h = q0_ref[...] + q1_ref[...]
    logits = jnp.dot(h, w_ref[...], preferred_element_type=jnp.float32)
    logits = logits + bout_ref[...]
    m = jnp.max(logits, axis=-1, keepdims=True)
    lse = jnp.log(jnp.sum(jnp.exp(logits - m), axis=-1, keepdims=True)) + m
    out_ref[...] = logits - lse


def _tc_out(q0, q1, w_out_t, b_out):
    return pl.pallas_call(
        _out_body,
        grid=(1,),
        in_specs=[pl.BlockSpec((V, S), lambda i: (0, 0)),
                  pl.BlockSpec((V, S), lambda i: (0, 0)),
                  pl.BlockSpec((S, C), lambda i: (0, 0)),
                  pl.BlockSpec((1, C), lambda i: (0, 0))],
        out_specs=pl.BlockSpec((V, C), lambda i: (0, 0)),
        out_shape=jax.ShapeDtypeStruct((V, C), jnp.float32),
    )(q0, q1, w_out_t, b_out)


def kernel(feat_Matrix, X_Node, X_Neis, dg_list, W_xi, b_xi, W_rou, b_rou,
           W_out, b_out):
    f32 = jnp.float32
    feat_pad = jnp.pad(feat_Matrix.astype(f32), ((0, V_PAD - V), (0, 0)))
    xn = jnp.pad(X_Node.astype(jnp.int32), (0, E_PAD - E), constant_values=V)
    xe = jnp.pad(X_Neis.astype(jnp.int32), (0, E_PAD - E), constant_values=V)
    xn3w = xn.reshape(NW, N_CH, CH)
    xe3w = xe.reshape(NW, N_CH, CH)
    xn3t = xn.reshape(NS, N_CHT, CH)
    dg = jnp.pad(dg_list.astype(f32), (0, E_PAD - E),
                 constant_values=1.0).reshape(E_PAD, 1)

    w1 = W_xi[:, :LN].T.astype(f32)
    w2 = W_xi[:, LN:].T.astype(f32)
    bxi = b_xi.reshape(1, S * S).astype(f32)
    w_rou_t = W_rou.T.astype(f32)
    brou = b_rou.reshape(1, S).astype(f32)
    w_out_t = W_out.T.astype(f32)
    bout = b_out.reshape(1, C).astype(f32)
    eye = jnp.eye(S, dtype=f32)
    q2c = jnp.tile(eye, (1, S))
    r2c = jnp.repeat(eye, S, axis=0)
    zeros_tile = jnp.zeros((ROWS_PER_TILE, S), dtype=f32)

    node_e, neis_e = _sc_gather_embeds(feat_pad, xn3w, xe3w)
    b_mat = _tc_b(node_e, w_rou_t, brou)
    hg = _sc_segsum_gather(b_mat, xn3t, zeros_tile)
    he = _tc_he(node_e, neis_e, hg, b_mat, dg, w1, w2, bxi, q2c, r2c)
    q2 = _sc_scatter_add(he, xn3t, zeros_tile)
    return _tc_out(q2[0, :V], q2[1, :V], w_out_t, bout)
